# Initial kernel scaffold; baseline (speedup 1.0000x reference)
import jax
import jax.numpy as jnp
from jax.experimental import pallas as pl


def kernel(input_ids, embed, W1, b1, W2, b2):
    raise NotImplementedError("write your pallas kernel here")


# speedup vs baseline: 43.1332x; 43.1332x over previous
import jax
import jax.numpy as jnp
from jax.experimental import pallas as pl

_L = 200
_V = 100
_VP = 128


def _tc_body(idsT_ref, emb_ref, w1_ref, b1_ref, w2_ref, b2_ref, out_ref):
    nb = idsT_ref.shape[1]
    iota_v = jax.lax.broadcasted_iota(jnp.int32, (_VP, nb), 0)

    def step(c, acc):
        blk = idsT_ref[pl.ds(pl.multiple_of(c * 8, 8), 8), :]
        for j in range(8):
            row = jnp.broadcast_to(blk[j : j + 1, :], (_VP, nb))
            acc = acc + (row == iota_v).astype(jnp.float32)
        return acc

    countsT = jax.lax.fori_loop(0, _L // 8, step, jnp.zeros((_VP, nb), jnp.float32))
    counts = countsT.T
    m1 = jnp.dot(emb_ref[...], w1_ref[...], preferred_element_type=jnp.float32)
    xw = jnp.dot(counts, m1, preferred_element_type=jnp.float32) * (1.0 / _L)
    h = jnp.maximum(xw + b1_ref[...], 0.0)
    out_ref[...] = jnp.dot(h, w2_ref[...], preferred_element_type=jnp.float32) + b2_ref[...]


def kernel(input_ids, embed, W1, b1, W2, b2):
    B, L = input_ids.shape
    NB = 2048
    ids_T = input_ids.astype(jnp.int32).T
    emb_pad = jnp.zeros((_VP, embed.shape[1]), jnp.float32).at[:_V].set(embed)
    out = pl.pallas_call(
        _tc_body,
        grid=(B // NB,),
        in_specs=[
            pl.BlockSpec((L, NB), lambda i: (0, i)),
            pl.BlockSpec((_VP, 16), lambda i: (0, 0)),
            pl.BlockSpec((16, 8), lambda i: (0, 0)),
            pl.BlockSpec((1, 8), lambda i: (0, 0)),
            pl.BlockSpec((8, _V), lambda i: (0, 0)),
            pl.BlockSpec((1, _V), lambda i: (0, 0)),
        ],
        out_specs=pl.BlockSpec((NB, _V), lambda i: (i, 0)),
        out_shape=jax.ShapeDtypeStruct((B, _V), jnp.float32),
    )(ids_T, emb_pad, W1, b1.reshape(1, -1), W2, b2.reshape(1, -1))
    return out


# speedup vs baseline: 61.0955x; 1.4164x over previous
import functools

import jax
import jax.numpy as jnp
from jax import lax
from jax.experimental import pallas as pl
from jax.experimental.pallas import tpu as pltpu
from jax.experimental.pallas import tpu_sc as plsc

_L = 200
_V = 100
_NC = 2
_NS = 16
_NW = _NC * _NS
_RB = 16


def _sc_hist(idsT_hbm, out_hbm, ids_buf, cnt_ref):
    c = lax.axis_index("c")
    s = lax.axis_index("s")
    wid = s * _NC + c
    nblk = idsT_hbm.shape[0]
    nblk_per = nblk // _NW
    iota16 = lax.iota(jnp.int32, 16)
    ones16 = jnp.ones((16,), jnp.float32)
    zeros16 = jnp.zeros((16,), jnp.float32)

    def blk_body(g, carry):
        blk = wid * nblk_per + g
        pltpu.sync_copy(idsT_hbm.at[blk], ids_buf)
        for i in range(_V):
            cnt_ref[i] = zeros16

        def l_body(l, carry2):
            idv = ids_buf[l]
            plsc.addupdate_scatter(cnt_ref, [idv, iota16], ones16)
            return carry2

        lax.fori_loop(0, _L, l_body, 0)
        pltpu.sync_copy(cnt_ref, out_hbm.at[blk])
        return carry

    lax.fori_loop(0, nblk_per, blk_body, 0)


def _tc_mlp(cnt_ref, emb_ref, w1_ref, b1_ref, w2_ref, b2_ref, out_ref):
    m1 = jnp.dot(emb_ref[...], w1_ref[...], preferred_element_type=jnp.float32)
    xw = jnp.dot(cnt_ref[...], m1, preferred_element_type=jnp.float32) * (1.0 / _L)
    h = jnp.maximum(xw + b1_ref[...], 0.0)
    out_ref[...] = jnp.dot(h, w2_ref[...], preferred_element_type=jnp.float32) + b2_ref[...]


def kernel(input_ids, embed, W1, b1, W2, b2):
    B, L = input_ids.shape
    nblk = B // _RB
    idsT3 = input_ids.astype(jnp.int32).reshape(nblk, _RB, L).transpose(0, 2, 1)

    mesh = plsc.VectorSubcoreMesh(
        core_axis_name="c", subcore_axis_name="s", num_cores=_NC, num_subcores=_NS
    )
    hist = pl.kernel(
        _sc_hist,
        out_type=jax.ShapeDtypeStruct((nblk, _V, _RB), jnp.float32),
        mesh=mesh,
        scratch_types=[
            pltpu.VMEM((L, _RB), jnp.int32),
            pltpu.VMEM((_V, _RB), jnp.float32),
        ],
        compiler_params=pltpu.CompilerParams(needs_layout_passes=False),
    )
    counts3 = hist(idsT3)
    counts = counts3.transpose(0, 2, 1).reshape(B, _V)

    NB = 2048
    out = pl.pallas_call(
        _tc_mlp,
        grid=(B // NB,),
        in_specs=[
            pl.BlockSpec((NB, _V), lambda i: (i, 0)),
            pl.BlockSpec((_V, 16), lambda i: (0, 0)),
            pl.BlockSpec((16, 8), lambda i: (0, 0)),
            pl.BlockSpec((1, 8), lambda i: (0, 0)),
            pl.BlockSpec((8, _V), lambda i: (0, 0)),
            pl.BlockSpec((1, _V), lambda i: (0, 0)),
        ],
        out_specs=pl.BlockSpec((NB, _V), lambda i: (i, 0)),
        out_shape=jax.ShapeDtypeStruct((B, _V), jnp.float32),
    )(counts, embed, W1, b1.reshape(1, -1), W2, b2.reshape(1, -1))
    return out


# speedup vs baseline: 94.5617x; 1.5478x over previous
import jax
import jax.numpy as jnp
from jax import lax
from jax.experimental import pallas as pl
from jax.experimental.pallas import tpu as pltpu
from jax.experimental.pallas import tpu_sc as plsc

_L = 200
_V = 100
_VS = 101
_NC = 2
_NS = 16
_NW = _NC * _NS
_RB = 16


def _sc_hist(idsT_hbm, out_hbm, ids0, ids1, cnt0, cnt1, sin0, sin1, sout0, sout1):
    c = lax.axis_index("c")
    s = lax.axis_index("s")
    wid = s * _NC + c
    nblk = idsT_hbm.shape[0]
    nper = nblk // _NW
    npair = nper // 2
    base = wid * nper
    rowbase = lax.iota(jnp.int32, 16) * jnp.int32(_VS)
    ones16 = jnp.ones((16,), jnp.float32)
    zeros16 = jnp.zeros((16,), jnp.float32)

    def in_cp(blk, buf, sem):
        return pltpu.make_async_copy(idsT_hbm.at[blk], buf, sem)

    def out_cp(blk, buf, sem):
        return pltpu.make_async_copy(buf, out_hbm.at[blk], sem)

    def hist(ids_buf, cnt_ref):
        for i in range(_VS):
            cnt_ref[pl.ds(16 * i, 16)] = zeros16

        def lb(l, carry):
            idv = ids_buf[l]
            plsc.addupdate_scatter(cnt_ref, [rowbase + idv], ones16)
            return carry

        lax.fori_loop(0, _L, lb, 0, unroll=8)

    in_cp(base + 0, ids0, sin0).start()
    in_cp(base + 1, ids1, sin1).start()

    def pair(g, carry):
        blk_a = base + 2 * g
        blk_b = blk_a + 1

        in_cp(blk_a, ids0, sin0).wait()

        @pl.when(g > 0)
        def _():
            out_cp(blk_a, cnt0, sout0).wait()

        hist(ids0, cnt0)
        out_cp(blk_a, cnt0, sout0).start()

        @pl.when(g + 1 < npair)
        def _():
            in_cp(blk_a + 2, ids0, sin0).start()

        in_cp(blk_b, ids1, sin1).wait()

        @pl.when(g > 0)
        def _():
            out_cp(blk_b, cnt1, sout1).wait()

        hist(ids1, cnt1)
        out_cp(blk_b, cnt1, sout1).start()

        @pl.when(g + 1 < npair)
        def _():
            in_cp(blk_b + 2, ids1, sin1).start()

        return carry

    lax.fori_loop(0, npair, pair, 0)
    out_cp(base, cnt0, sout0).wait()
    out_cp(base, cnt1, sout1).wait()


def _tc_mlp(cnt_ref, emb_ref, w1_ref, b1_ref, w2_ref, b2_ref, out_ref):
    m1 = jnp.dot(emb_ref[...], w1_ref[...], preferred_element_type=jnp.float32)
    xw = jnp.dot(cnt_ref[...], m1, preferred_element_type=jnp.float32) * (1.0 / _L)
    h = jnp.maximum(xw + b1_ref[...], 0.0)
    out_ref[...] = jnp.dot(h, w2_ref[...], preferred_element_type=jnp.float32) + b2_ref[...]


def kernel(input_ids, embed, W1, b1, W2, b2):
    B, L = input_ids.shape
    nblk = B // _RB
    idsT3 = input_ids.astype(jnp.int32).reshape(nblk, _RB, L).transpose(0, 2, 1)

    mesh = plsc.VectorSubcoreMesh(
        core_axis_name="c", subcore_axis_name="s", num_cores=_NC, num_subcores=_NS
    )
    hist = pl.kernel(
        _sc_hist,
        out_type=jax.ShapeDtypeStruct((nblk, _RB * _VS), jnp.float32),
        mesh=mesh,
        scratch_types=[
            pltpu.VMEM((L, _RB), jnp.int32),
            pltpu.VMEM((L, _RB), jnp.int32),
            pltpu.VMEM((_RB * _VS,), jnp.float32),
            pltpu.VMEM((_RB * _VS,), jnp.float32),
            pltpu.SemaphoreType.DMA,
            pltpu.SemaphoreType.DMA,
            pltpu.SemaphoreType.DMA,
            pltpu.SemaphoreType.DMA,
        ],
        compiler_params=pltpu.CompilerParams(needs_layout_passes=False),
    )
    counts = hist(idsT3).reshape(B, _VS)

    emb_pad = jnp.zeros((_VS, embed.shape[1]), jnp.float32).at[:_V].set(embed)
    NB = 2048
    out = pl.pallas_call(
        _tc_mlp,
        grid=(B // NB,),
        in_specs=[
            pl.BlockSpec((NB, _VS), lambda i: (i, 0)),
            pl.BlockSpec((_VS, 16), lambda i: (0, 0)),
            pl.BlockSpec((16, 8), lambda i: (0, 0)),
            pl.BlockSpec((1, 8), lambda i: (0, 0)),
            pl.BlockSpec((8, _V), lambda i: (0, 0)),
            pl.BlockSpec((1, _V), lambda i: (0, 0)),
        ],
        out_specs=pl.BlockSpec((NB, _V), lambda i: (i, 0)),
        out_shape=jax.ShapeDtypeStruct((B, _V), jnp.float32),
    )(counts, emb_pad, W1, b1.reshape(1, -1), W2, b2.reshape(1, -1))
    return out


# speedup vs baseline: 113.6244x; 1.2016x over previous
import jax
import jax.numpy as jnp
from jax import lax
from jax.experimental import pallas as pl
from jax.experimental.pallas import tpu as pltpu
from jax.experimental.pallas import tpu_sc as plsc

_L = 200
_LS = 205
_V = 100
_VS = 101
_VP = 128
_NC = 2
_NS = 16
_NW = _NC * _NS
_RB = 16
_QD = 4


def _sc_hist(ids_hbm, out_hbm, ids0, ids1, ids2, ids3, cnt_s, cnt_o,
             sin0, sin1, sin2, sin3, sout):
    c = lax.axis_index("c")
    s = lax.axis_index("s")
    wid = s * _NC + c
    nblk = ids_hbm.shape[0] // _RB
    nper = nblk // _NW
    nquad = nper // _QD
    base = wid * nper
    iota16 = lax.iota(jnp.int32, 16)
    rowbase = iota16 * jnp.int32(_VS)
    ones16 = jnp.ones((16,), jnp.float32)
    zeros16 = jnp.zeros((16,), jnp.float32)
    ids_bufs = (ids0, ids1, ids2, ids3)
    sins = (sin0, sin1, sin2, sin3)

    def in_cp(blk, j):
        return pltpu.make_async_copy(
            ids_hbm.at[pl.ds(blk * _RB, _RB), :], ids_bufs[j], sins[j]
        )

    def out_cp(qblk):
        return pltpu.make_async_copy(
            cnt_o, out_hbm.at[pl.ds(qblk * _RB * _VP, _QD * _RB * _VP)], sout
        )

    def hist(ids_buf):
        for i in range(_RB * _VS // 16 + 1):
            cnt_s[pl.ds(16 * i, 16)] = zeros16

        def outer(g2, basev):
            for j in range(16):
                pv = basev + (jnp.full((16,), j, jnp.int32) ^ iota16)
                idv = plsc.load_gather(ids_buf, [iota16, pv])
                plsc.addupdate_scatter(cnt_s, [rowbase + idv], ones16)
            return basev + 16

        basev = lax.fori_loop(0, (_L // 16), outer, jnp.zeros((16,), jnp.int32))
        for j in range(_L % 16):
            pv = basev + (jnp.full((16,), j, jnp.int32) ^ (iota16 & 7))
            idv = plsc.load_gather(ids_buf, [iota16, pv])
            plsc.addupdate_scatter(cnt_s, [rowbase + idv], ones16)

    def relayout(j):
        for r in range(_RB):
            for q in range(7):
                v = plsc.load_gather(cnt_s, [iota16 + (r * _VS + 16 * q)])
                cnt_o[pl.ds(j * _RB * _VP + r * _VP + 16 * q, 16)] = v

    for j in range(_QD):
        in_cp(base + j, j).start()

    def quad(g, carry):
        qstart = base + _QD * g
        for j in range(_QD):
            in_cp(qstart + j, j).wait()
            hist(ids_bufs[j])
            if j == 0:
                @pl.when(g > 0)
                def _():
                    out_cp(base).wait()
            relayout(j)

            @pl.when(g < nquad - 1)
            def _():
                in_cp(qstart + j + _QD, j).start()

        out_cp(qstart).start()
        return carry

    lax.fori_loop(0, nquad, quad, 0)
    out_cp(base).wait()


def _tc_mlp(cnt_ref, emb_ref, w1_ref, b1_ref, w2_ref, b2_ref, out_ref):
    m1 = jnp.dot(emb_ref[...], w1_ref[...], preferred_element_type=jnp.float32)
    xw = jnp.dot(cnt_ref[...], m1, preferred_element_type=jnp.float32) * (1.0 / _L)
    h = jnp.maximum(xw + b1_ref[...], 0.0)
    out_ref[...] = jnp.dot(h, w2_ref[...], preferred_element_type=jnp.float32) + b2_ref[...]


def kernel(input_ids, embed, W1, b1, W2, b2):
    B, L = input_ids.shape
    nblk = B // _RB
    ids = input_ids.astype(jnp.int32)

    mesh = plsc.VectorSubcoreMesh(
        core_axis_name="c", subcore_axis_name="s", num_cores=_NC, num_subcores=_NS
    )
    hist = pl.kernel(
        _sc_hist,
        out_type=jax.ShapeDtypeStruct((B * _VP,), jnp.float32),
        mesh=mesh,
        scratch_types=[
            pltpu.VMEM((_RB, _L), jnp.int32),
            pltpu.VMEM((_RB, _L), jnp.int32),
            pltpu.VMEM((_RB, _L), jnp.int32),
            pltpu.VMEM((_RB, _L), jnp.int32),
            pltpu.VMEM((_RB * _VS + 16,), jnp.float32),
            pltpu.VMEM((_QD * _RB * _VP,), jnp.float32),
            pltpu.SemaphoreType.DMA,
            pltpu.SemaphoreType.DMA,
            pltpu.SemaphoreType.DMA,
            pltpu.SemaphoreType.DMA,
            pltpu.SemaphoreType.DMA,
        ],
        compiler_params=pltpu.CompilerParams(needs_layout_passes=False),
    )
    counts = hist(ids).reshape(B, _VP)

    emb_pad = jnp.zeros((_VP, embed.shape[1]), jnp.float32).at[:_V].set(embed)
    NB = 2048
    out = pl.pallas_call(
        _tc_mlp,
        grid=(B // NB,),
        in_specs=[
            pl.BlockSpec((NB, _VP), lambda i: (i, 0)),
            pl.BlockSpec((_VP, 16), lambda i: (0, 0)),
            pl.BlockSpec((16, 8), lambda i: (0, 0)),
            pl.BlockSpec((1, 8), lambda i: (0, 0)),
            pl.BlockSpec((8, _V), lambda i: (0, 0)),
            pl.BlockSpec((1, _V), lambda i: (0, 0)),
        ],
        out_specs=pl.BlockSpec((NB, _V), lambda i: (i, 0)),
        out_shape=jax.ShapeDtypeStruct((B, _V), jnp.float32),
    )(counts, emb_pad, W1, b1.reshape(1, -1), W2, b2.reshape(1, -1))
    return out
